# baseline (speedup 1.0000x reference)
import jax
import jax.numpy as jnp
from jax import lax
from jax.experimental import pallas as pl
from jax.experimental.pallas import tpu as pltpu
from jax.experimental.pallas import tpu_sc as plsc

_N = 100000
_LL = 10
_ED = 16
_OUT_D = _ED + 2 + _LL

_NW = 32
_NC = 4
_CNT = 784
_SPAN = _NC * _CNT


def _body(an_hbm, nu_hbm, l_hbm, llt_hbm, tab_hbm, out_hbm,
          idx_vs, tab_v, out_vs, sem_a, sem, sem_o):
    wid = lax.axis_index("s") * 2 + lax.axis_index("c")
    t = wid * (_N // _NW)
    wbase = pl.multiple_of(jnp.minimum(t - lax.rem(t, 8), _N - _SPAN), 8)

    d_tab = pltpu.async_copy(tab_hbm, tab_v, sem)

    iota = lax.iota(jnp.int32, 16)
    rowc = [jnp.full((16,), f, jnp.int32) for f in range(_ED)]

    bases = [pl.multiple_of(wbase + k * _CNT, 8) for k in range(_NC)]
    an_descs = [
        pltpu.async_copy(an_hbm.at[pl.ds(bases[k], _CNT)], idx_vs[k], sem_a)
        for k in range(_NC)
    ]
    in_descs = []
    for k in range(_NC):
        in_descs.append([
            pltpu.async_copy(nu_hbm.at[:, pl.ds(bases[k], _CNT)],
                             out_vs[k].at[pl.ds(_ED, 1)], sem),
            pltpu.async_copy(l_hbm.at[:, pl.ds(bases[k], _CNT)],
                             out_vs[k].at[pl.ds(_ED + 1, 1)], sem),
            pltpu.async_copy(llt_hbm.at[:, pl.ds(bases[k], _CNT)],
                             out_vs[k].at[pl.ds(_ED + 2, _LL)], sem),
        ])
    d_tab.wait()

    out_descs = [None] * _NC
    for k in range(_NC):
        out_v = out_vs[k]
        idx_v = idx_vs[k]
        an_descs[k].wait()

        @plsc.parallel_loop(0, _CNT, step=16, unroll=2)
        def vec_body(i):
            rows = iota + i
            an = plsc.load_gather(idx_v, [rows])
            for f in range(_ED):
                v = plsc.load_gather(tab_v, [an, rowc[f]])
                plsc.store_scatter(out_v, [rowc[f], rows], v)

        for d in in_descs[k]:
            d.wait()

        out_descs[k] = pltpu.async_copy(
            out_v, out_hbm.at[:, pl.ds(bases[k], _CNT)], sem_o
        )

    for d in out_descs:
        d.wait()


@jax.jit
def _node_feat(an, nu2, l2, llt, tab):
    mesh = plsc.VectorSubcoreMesh(core_axis_name="c", subcore_axis_name="s")
    run = pl.kernel(
        _body,
        out_type=jax.ShapeDtypeStruct((_OUT_D, _N), jnp.float32),
        mesh=mesh,
        scratch_types=[
            [pltpu.VMEM((_CNT,), jnp.int32) for _ in range(_NC)],
            pltpu.VMEM((10, _ED), jnp.float32),
            [pltpu.VMEM((_OUT_D, _CNT), jnp.float32) for _ in range(_NC)],
            pltpu.SemaphoreType.DMA,
            pltpu.SemaphoreType.DMA,
            pltpu.SemaphoreType.DMA,
        ],
        compiler_params=pltpu.CompilerParams(
            use_tc_tiling_on_sc=False,
            needs_layout_passes=False,
            disable_bounds_checks=True,
            disable_semaphore_checks=True,
            skip_device_barrier=True,
        ),
    )
    return run(an, nu2, l2, llt, tab)


def kernel(atomic_numbers, max_nu, max_l, l_list, emb_table):
    out_t = _node_feat(
        atomic_numbers,
        max_nu.reshape(1, _N),
        max_l.reshape(1, _N),
        l_list.T,
        emb_table,
    )
    return out_t.T


# speedup vs baseline: 1.0213x; 1.0213x over previous
import jax
import jax.numpy as jnp
from jax import lax
from jax.experimental import pallas as pl
from jax.experimental.pallas import tpu as pltpu
from jax.experimental.pallas import tpu_sc as plsc

_N = 100000
_LL = 10
_ED = 16
_OUT_D = _ED + 2 + _LL

_NW = 32
_NC = 2
_CNT = 1568
_SPAN = _NC * _CNT


def _body(an_hbm, nu_hbm, l_hbm, llt_hbm, tab_hbm, out_hbm,
          idx_vs, tab_v, out_vs, sem_a, sem, sem_o):
    wid = lax.axis_index("s") * 2 + lax.axis_index("c")
    t = wid * (_N // _NW)
    wbase = pl.multiple_of(jnp.minimum(t - lax.rem(t, 8), _N - _SPAN), 8)

    d_tab = pltpu.async_copy(tab_hbm, tab_v, sem)

    iota = lax.iota(jnp.int32, 16)
    rowc = [jnp.full((16,), f, jnp.int32) for f in range(_ED)]

    bases = [pl.multiple_of(wbase + k * _CNT, 8) for k in range(_NC)]
    an_descs = [
        pltpu.async_copy(an_hbm.at[pl.ds(bases[k], _CNT)], idx_vs[k], sem_a)
        for k in range(_NC)
    ]
    in_descs = []
    for k in range(_NC):
        in_descs.append([
            pltpu.async_copy(nu_hbm.at[:, pl.ds(bases[k], _CNT)],
                             out_vs[k].at[pl.ds(_ED, 1)], sem),
            pltpu.async_copy(l_hbm.at[:, pl.ds(bases[k], _CNT)],
                             out_vs[k].at[pl.ds(_ED + 1, 1)], sem),
            pltpu.async_copy(llt_hbm.at[:, pl.ds(bases[k], _CNT)],
                             out_vs[k].at[pl.ds(_ED + 2, _LL)], sem),
        ])
    d_tab.wait()

    out_descs = [None] * _NC
    for k in range(_NC):
        out_v = out_vs[k]
        idx_v = idx_vs[k]
        an_descs[k].wait()

        @plsc.parallel_loop(0, _CNT, step=16, unroll=1)
        def vec_body(i):
            rows = iota + i
            an = plsc.load_gather(idx_v, [rows])
            for f in range(_ED):
                v = plsc.load_gather(tab_v, [an, rowc[f]])
                plsc.store_scatter(out_v, [rowc[f], rows], v)

        for d in in_descs[k]:
            d.wait()

        out_descs[k] = pltpu.async_copy(
            out_v, out_hbm.at[:, pl.ds(bases[k], _CNT)], sem_o
        )

    for d in out_descs:
        d.wait()


@jax.jit
def _node_feat(an, nu2, l2, llt, tab):
    mesh = plsc.VectorSubcoreMesh(core_axis_name="c", subcore_axis_name="s")
    run = pl.kernel(
        _body,
        out_type=jax.ShapeDtypeStruct((_OUT_D, _N), jnp.float32),
        mesh=mesh,
        scratch_types=[
            [pltpu.VMEM((_CNT,), jnp.int32) for _ in range(_NC)],
            pltpu.VMEM((10, _ED), jnp.float32),
            [pltpu.VMEM((_OUT_D, _CNT), jnp.float32) for _ in range(_NC)],
            pltpu.SemaphoreType.DMA,
            pltpu.SemaphoreType.DMA,
            pltpu.SemaphoreType.DMA,
        ],
        compiler_params=pltpu.CompilerParams(
            use_tc_tiling_on_sc=False,
            needs_layout_passes=False,
            disable_bounds_checks=True,
            disable_semaphore_checks=True,
            skip_device_barrier=True,
        ),
    )
    return run(an, nu2, l2, llt, tab)


def kernel(atomic_numbers, max_nu, max_l, l_list, emb_table):
    out_t = _node_feat(
        atomic_numbers,
        max_nu.reshape(1, _N),
        max_l.reshape(1, _N),
        l_list.T,
        emb_table,
    )
    return out_t.T
